# baseline (speedup 1.0000x reference)
import functools

import jax
import jax.numpy as jnp
from jax import lax
from jax.experimental import pallas as pl
from jax.experimental.pallas import tpu as pltpu
from jax.experimental.pallas import tpu_sc as plsc

_KEEP = 256
_L = 1024
_D = 384


def _ladder8(a):
    b = a[0:4] + a[4:8]
    c = b[0:2] + b[2:4]
    return c[0:1] + c[1:2]


def _tr_reduce(tile):
    T = jnp.transpose(tile, (1, 0))
    acc = T[0:8]
    for j in range(1, 16):
        acc = acc + T[8 * j:8 * j + 8]
    return _ladder8(acc)


def _red_partials(p):
    r0 = _tr_reduce(p[:, 0:128])
    r1 = _tr_reduce(p[:, 128:256])
    r2 = _tr_reduce(p[:, 256:384])
    return (r0 + r1) + r2


def _stats_kernel(x_ref, vsum_ref, nsq_ref):
    acc8 = jnp.zeros((8, _D), jnp.float32)
    rows = []
    for g in range(8):
        Xg = x_ref[0, 128 * g:128 * (g + 1), :]
        for t in range(16):
            acc8 = acc8 + Xg[8 * t:8 * t + 8]
        rows.append(_red_partials(Xg * Xg))
    v4 = acc8[0:4] + acc8[4:8]
    v2 = v4[0:2] + v4[2:4]
    vsum_ref[0] = v2[0:1] + v2[1:2]
    nsq_ref[0] = jnp.concatenate(rows, axis=1)


def _stats(x):
    B = x.shape[0]
    return pl.pallas_call(
        _stats_kernel,
        grid=(B,),
        in_specs=[pl.BlockSpec((1, _L, _D), lambda b: (b, 0, 0))],
        out_specs=[
            pl.BlockSpec((1, 1, _D), lambda b: (b, 0, 0)),
            pl.BlockSpec((1, 1, _L), lambda b: (b, 0, 0)),
        ],
        out_shape=[
            jax.ShapeDtypeStruct((B, 1, _D), jnp.float32),
            jax.ShapeDtypeStruct((B, 1, _L), jnp.float32),
        ],
    )(x)


def _sc_gather(table, idx):
    n = idx.shape[0]
    info = plsc.get_sparse_core_info()
    nc, ns = info.num_cores, info.num_subcores
    nw = nc * ns
    per_w = n // nw
    ch = 128

    mesh = plsc.VectorSubcoreMesh(core_axis_name="c", subcore_axis_name="s")

    @functools.partial(
        pl.kernel,
        mesh=mesh,
        out_type=jax.ShapeDtypeStruct((n, _D), jnp.float32),
        scratch_types=[
            pltpu.VMEM((ch,), jnp.int32),
            pltpu.VMEM((ch, _D), jnp.float32),
            pltpu.SemaphoreType.DMA,
        ],
    )
    def k(table_hbm, idx_hbm, out_hbm, idx_v, rows_v, sem):
        wid = lax.axis_index("s") * nc + lax.axis_index("c")
        base = wid * per_w

        def body(i, carry):
            off = base + i * ch
            pltpu.sync_copy(idx_hbm.at[pl.ds(off, ch)], idx_v)
            pltpu.async_copy(table_hbm.at[idx_v], rows_v, sem).wait()
            pltpu.sync_copy(rows_v, out_hbm.at[pl.ds(off, ch)])
            return carry

        lax.fori_loop(0, per_w // ch, body, 0)

    return k(table, idx)


def kernel(image_features, t_cls):
    x = image_features
    B = x.shape[0]
    eps = jnp.float32(1e-12)

    vsum, nsq = _stats(x)
    v_cls = vsum[:, 0, :] * jnp.float32(1.0 / 1024.0)
    m_cls = (v_cls + t_cls)[:, None, :]
    nm = jnp.linalg.norm(m_cls, ord=2, axis=-1, keepdims=True)
    m_cls_normalized = m_cls / jnp.maximum(nm, eps)

    norms = jnp.maximum(jnp.sqrt(nsq[:, 0, :]), eps)
    image_norm_normalized = x / norms[:, :, None]

    logits = jnp.einsum('bqd,bld->bql', m_cls_normalized,
                        image_norm_normalized)
    scort = jax.nn.softmax(logits, axis=-1)[:, 0, :]
    _, top_indices = jax.lax.top_k(scort, _KEEP)

    flat = (top_indices
            + (jnp.arange(B, dtype=jnp.int32) * _L)[:, None]).reshape(-1)
    Z_R = _sc_gather(x.reshape(B * _L, _D), flat)
    return Z_R.reshape(B, _KEEP, _D)


# speedup vs baseline: 1.0216x; 1.0216x over previous
import functools

import jax
import jax.numpy as jnp
from jax import lax
from jax.experimental import pallas as pl
from jax.experimental.pallas import tpu as pltpu
from jax.experimental.pallas import tpu_sc as plsc

_KEEP = 256
_L = 1024
_D = 384


def _ladder8(a):
    b = a[0:4] + a[4:8]
    c = b[0:2] + b[2:4]
    return c[0:1] + c[1:2]


def _tr_reduce(tile):
    T = jnp.transpose(tile, (1, 0))
    acc = T[0:8]
    for j in range(1, 16):
        acc = acc + T[8 * j:8 * j + 8]
    return _ladder8(acc)


def _red_partials(p):
    r0 = _tr_reduce(p[:, 0:128])
    r1 = _tr_reduce(p[:, 128:256])
    r2 = _tr_reduce(p[:, 256:384])
    return (r0 + r1) + r2


def _stats_kernel(x_ref, vsum_ref, nsq_ref):
    acc8 = jnp.zeros((8, _D), jnp.float32)
    rows = []
    for g in range(8):
        Xg = x_ref[0, 128 * g:128 * (g + 1), :]
        for t in range(16):
            acc8 = acc8 + Xg[8 * t:8 * t + 8]
        rows.append(_red_partials(Xg * Xg))
    v4 = acc8[0:4] + acc8[4:8]
    v2 = v4[0:2] + v4[2:4]
    vsum_ref[0] = v2[0:1] + v2[1:2]
    nsq_ref[0] = jnp.concatenate(rows, axis=1)


def _stats(x):
    B = x.shape[0]
    return pl.pallas_call(
        _stats_kernel,
        grid=(B,),
        in_specs=[pl.BlockSpec((1, _L, _D), lambda b: (b, 0, 0))],
        out_specs=[
            pl.BlockSpec((1, 1, _D), lambda b: (b, 0, 0)),
            pl.BlockSpec((1, 1, _L), lambda b: (b, 0, 0)),
        ],
        out_shape=[
            jax.ShapeDtypeStruct((B, 1, _D), jnp.float32),
            jax.ShapeDtypeStruct((B, 1, _L), jnp.float32),
        ],
    )(x)


def _sc_gather(table, idx):
    n = idx.shape[0]
    info = plsc.get_sparse_core_info()
    nc, ns = info.num_cores, info.num_subcores
    nw = nc * ns
    per_w = n // nw
    ch = 128

    nchunk = per_w // ch
    mesh = plsc.VectorSubcoreMesh(core_axis_name="c", subcore_axis_name="s")

    @functools.partial(
        pl.kernel,
        mesh=mesh,
        out_type=jax.ShapeDtypeStruct((n, _D), jnp.float32),
        scratch_types=[
            pltpu.VMEM((per_w,), jnp.int32),
            pltpu.VMEM((ch, _D), jnp.float32),
            pltpu.VMEM((ch, _D), jnp.float32),
            pltpu.SemaphoreType.DMA,
            pltpu.SemaphoreType.DMA,
        ],
    )
    def k(table_hbm, idx_hbm, out_hbm, idx_v, r0, r1, s0, s1):
        wid = lax.axis_index("s") * nc + lax.axis_index("c")
        base = wid * per_w
        pltpu.sync_copy(idx_hbm.at[pl.ds(base, per_w)], idx_v)
        bufs = (r0, r1)
        sems = (s0, s1)
        copies = [None] * nchunk
        copies[0] = pltpu.async_copy(
            table_hbm.at[idx_v.at[pl.ds(0, ch)]], bufs[0], sems[0])
        for i in range(nchunk):
            if i + 1 < nchunk:
                copies[i + 1] = pltpu.async_copy(
                    table_hbm.at[idx_v.at[pl.ds((i + 1) * ch, ch)]],
                    bufs[(i + 1) % 2], sems[(i + 1) % 2])
            copies[i].wait()
            pltpu.sync_copy(bufs[i % 2], out_hbm.at[pl.ds(base + i * ch, ch)])

    return k(table, idx)


def kernel(image_features, t_cls):
    x = image_features
    B = x.shape[0]
    eps = jnp.float32(1e-12)

    vsum, nsq = _stats(x)
    v_cls = vsum[:, 0, :] * jnp.float32(1.0 / 1024.0)
    m_cls = (v_cls + t_cls)[:, None, :]
    nm = jnp.linalg.norm(m_cls, ord=2, axis=-1, keepdims=True)
    m_cls_normalized = m_cls / jnp.maximum(nm, eps)

    norms = jnp.maximum(jnp.sqrt(nsq[:, 0, :]), eps)
    image_norm_normalized = x / norms[:, :, None]

    logits = jnp.einsum('bqd,bld->bql', m_cls_normalized,
                        image_norm_normalized)
    scort = jax.nn.softmax(logits, axis=-1)[:, 0, :]
    _, top_indices = jax.lax.top_k(scort, _KEEP)

    flat = (top_indices
            + (jnp.arange(B, dtype=jnp.int32) * _L)[:, None]).reshape(-1)
    Z_R = _sc_gather(x.reshape(B * _L, _D), flat)
    return Z_R.reshape(B, _KEEP, _D)


# speedup vs baseline: 1.0221x; 1.0005x over previous
import functools

import jax
import jax.numpy as jnp
from jax import lax
from jax.experimental import pallas as pl
from jax.experimental.pallas import tpu as pltpu
from jax.experimental.pallas import tpu_sc as plsc

_KEEP = 256
_L = 1024
_D = 384


def _ladder8(a):
    b = a[0:4] + a[4:8]
    c = b[0:2] + b[2:4]
    return c[0:1] + c[1:2]


def _tr_reduce(tile):
    T = jnp.transpose(tile, (1, 0))
    acc = T[0:8]
    for j in range(1, 16):
        acc = acc + T[8 * j:8 * j + 8]
    return _ladder8(acc)


def _red_partials(p):
    r0 = _tr_reduce(p[:, 0:128])
    r1 = _tr_reduce(p[:, 128:256])
    r2 = _tr_reduce(p[:, 256:384])
    return (r0 + r1) + r2


def _stats_kernel(x_ref, vsum_ref, nsq_ref):
    acc8 = jnp.zeros((8, _D), jnp.float32)
    rows = []
    for g in range(8):
        Xg = x_ref[0, 128 * g:128 * (g + 1), :]
        for t in range(16):
            acc8 = acc8 + Xg[8 * t:8 * t + 8]
        rows.append(_red_partials(Xg * Xg))
    v4 = acc8[0:4] + acc8[4:8]
    v2 = v4[0:2] + v4[2:4]
    vsum_ref[0] = v2[0:1] + v2[1:2]
    nsq_ref[0] = jnp.concatenate(rows, axis=1)


def _stats(x):
    B = x.shape[0]
    return pl.pallas_call(
        _stats_kernel,
        grid=(B,),
        in_specs=[pl.BlockSpec((1, _L, _D), lambda b: (b, 0, 0))],
        out_specs=[
            pl.BlockSpec((1, 1, _D), lambda b: (b, 0, 0)),
            pl.BlockSpec((1, 1, _L), lambda b: (b, 0, 0)),
        ],
        out_shape=[
            jax.ShapeDtypeStruct((B, 1, _D), jnp.float32),
            jax.ShapeDtypeStruct((B, 1, _L), jnp.float32),
        ],
    )(x)


def _sc_gather(table, idx):
    n = idx.shape[0]
    info = plsc.get_sparse_core_info()
    nc, ns = info.num_cores, info.num_subcores
    nw = nc * ns
    per_w = n // nw
    ch = 128

    nchunk = per_w // ch
    mesh = plsc.VectorSubcoreMesh(core_axis_name="c", subcore_axis_name="s")

    @functools.partial(
        pl.kernel,
        mesh=mesh,
        out_type=jax.ShapeDtypeStruct((n, _D), jnp.float32),
        scratch_types=[
            pltpu.VMEM((per_w,), jnp.int32),
            pltpu.VMEM((ch, _D), jnp.float32),
            pltpu.VMEM((ch, _D), jnp.float32),
            pltpu.SemaphoreType.DMA,
            pltpu.SemaphoreType.DMA,
            pltpu.SemaphoreType.DMA,
            pltpu.SemaphoreType.DMA,
        ],
    )
    def k(table_hbm, idx_hbm, out_hbm, idx_v, r0, r1, s0, s1, w0, w1):
        wid = lax.axis_index("s") * nc + lax.axis_index("c")
        base = wid * per_w
        pltpu.sync_copy(idx_hbm.at[pl.ds(base, per_w)], idx_v)
        bufs = (r0, r1)
        sems = (s0, s1)
        wsems = (w0, w1)
        gathers = [None] * nchunk
        writes = [None] * nchunk
        gathers[0] = pltpu.async_copy(
            table_hbm.at[idx_v.at[pl.ds(0, ch)]], bufs[0], sems[0])
        for i in range(nchunk):
            if i + 1 < nchunk:
                if i >= 1:
                    writes[i - 1].wait()
                gathers[i + 1] = pltpu.async_copy(
                    table_hbm.at[idx_v.at[pl.ds((i + 1) * ch, ch)]],
                    bufs[(i + 1) % 2], sems[(i + 1) % 2])
            gathers[i].wait()
            writes[i] = pltpu.async_copy(
                bufs[i % 2], out_hbm.at[pl.ds(base + i * ch, ch)],
                wsems[i % 2])
        writes[nchunk - 1].wait()
        writes[nchunk - 2].wait()

    return k(table, idx)


def kernel(image_features, t_cls):
    x = image_features
    B = x.shape[0]
    eps = jnp.float32(1e-12)

    vsum, nsq = _stats(x)
    v_cls = vsum[:, 0, :] * jnp.float32(1.0 / 1024.0)
    m_cls = (v_cls + t_cls)[:, None, :]
    nm = jnp.linalg.norm(m_cls, ord=2, axis=-1, keepdims=True)
    m_cls_normalized = m_cls / jnp.maximum(nm, eps)

    norms = jnp.maximum(jnp.sqrt(nsq[:, 0, :]), eps)
    image_norm_normalized = x / norms[:, :, None]

    logits = jnp.einsum('bqd,bld->bql', m_cls_normalized,
                        image_norm_normalized)
    scort = jax.nn.softmax(logits, axis=-1)[:, 0, :]
    _, top_indices = jax.lax.top_k(scort, _KEEP)

    flat = (top_indices
            + (jnp.arange(B, dtype=jnp.int32) * _L)[:, None]).reshape(-1)
    Z_R = _sc_gather(x.reshape(B * _L, _D), flat)
    return Z_R.reshape(B, _KEEP, _D)
